# baseline (speedup 1.0000x reference)
import functools

import jax
import jax.numpy as jnp
from jax import lax
from jax.experimental import pallas as pl
from jax.experimental.pallas import tpu as pltpu
from jax.experimental.pallas import tpu_sc as plsc

CNT_E = 50000
CNT_R = 256
DIM = 512
MAX_NEIGHBOR = 16
NUM_NEG = 2
B = 1024
MARGIN = 1.0

NW = 32
N_ENC = 3 * B * 2
N_T2 = 9 * B
N_R1 = 3 * B
ENC_PER = N_ENC // NW
T2_PER = N_T2 // NW
R1_PER = N_R1 // NW
CH_A = 48
GE = 3
ACC_N = 48
GW = ACC_N // GE
NWIN = ENC_PER // ACC_N


def _sc_rows_body(ids_t2, ids_r1, e_emb, r_emb, rows_t2_o, rows_r1_o,
                  idx0, idx1, row0, row1, sem0, sem1):
    wid = lax.axis_index("s") * 2 + lax.axis_index("c")
    idxb = (idx0, idx1)
    rowb = (row0, row1)
    semb = (sem0, sem1)
    t2_base = wid * T2_PER
    r1_base = wid * R1_PER
    chunks = []
    for c in range(T2_PER // CH_A):
        chunks.append((ids_t2, e_emb, rows_t2_o, t2_base + c * CH_A))
    for c in range(R1_PER // CH_A):
        chunks.append((ids_r1, r_emb, rows_r1_o, r1_base + c * CH_A))
    descs = []
    for c, (src_ids, table, dst, base) in enumerate(chunks):
        p = c & 1
        pltpu.sync_copy(src_ids.at[pl.ds(base, CH_A)], idxb[p])
        descs.append(pltpu.async_copy(table.at[idxb[p]], rowb[p], semb[p]))
        if c > 0:
            descs[c - 1].wait()
            _, _, pdst, pbase = chunks[c - 1]
            pltpu.sync_copy(rowb[1 - p], pdst.at[pl.ds(pbase, CH_A)])
    descs[-1].wait()
    _, _, pdst, pbase = chunks[-1]
    pltpu.sync_copy(rowb[(len(chunks) - 1) & 1], pdst.at[pl.ds(pbase, CH_A)])


def _sc_rows(ids_t2, ids_r1, e_emb, r_emb):
    mesh = plsc.VectorSubcoreMesh(core_axis_name="c", subcore_axis_name="s")
    f = pl.kernel(
        _sc_rows_body,
        out_type=(
            jax.ShapeDtypeStruct((N_T2, DIM), jnp.float32),
            jax.ShapeDtypeStruct((N_R1, DIM), jnp.float32),
        ),
        mesh=mesh,
        scratch_types=[
            pltpu.VMEM((CH_A,), jnp.int32),
            pltpu.VMEM((CH_A,), jnp.int32),
            pltpu.VMEM((CH_A, DIM), jnp.float32),
            pltpu.VMEM((CH_A, DIM), jnp.float32),
            pltpu.SemaphoreType.DMA,
            pltpu.SemaphoreType.DMA,
        ],
    )
    return f(ids_t2, ids_r1, e_emb, r_emb)


def _sc_enc_body(ids_enc, e_emb, nei_pack,
                 e_sum_o, rids_o,
                 idx0, idx1, row0, row1, acc_v, prow_v, rids_v, pidx_v,
                 sem0, sem1):
    wid = lax.axis_index("s") * 2 + lax.axis_index("c")
    idxb = (idx0, idx1)
    rowb = (row0, row1)
    semb = (sem0, sem1)
    enc_base = wid * ENC_PER


    def reduce_group(buf, g2):
        for k in range(GE):
            slot = g2 * GE + k

            def col_body(c, cc, k=k, slot=slot):
                vals = [buf[k * 16 + r, pl.ds(c * 16, 16)] for r in range(16)]
                while len(vals) > 1:
                    vals = [vals[i] + vals[i + 1]
                            for i in range(0, len(vals), 2)]
                acc_v[slot, pl.ds(c * 16, 16)] = vals[0]
                return cc

            lax.fori_loop(0, DIM // 16, col_body, 0)

    def start_group(w, g2, p):
        for k in range(GE):
            e = g2 * GE + k
            idxb[p][pl.ds(k * 16, 16)] = prow_v[e, pl.ds(0, 16)]
        return pltpu.async_copy(e_emb.at[idxb[p]], rowb[p], semb[p])

    def win_body(w, carry):
        base = enc_base + w * ACC_N
        pltpu.sync_copy(ids_enc.at[pl.ds(base, ACC_N)], pidx_v)
        pltpu.async_copy(nei_pack.at[pidx_v], prow_v, sem0).wait()

        def rid_body(e, c):
            rids_v[e] = prow_v[e, pl.ds(16, 16)]
            return c

        lax.fori_loop(0, ACC_N, rid_body, 0)
        pltpu.sync_copy(rids_v, rids_o.at[pl.ds(base, ACC_N)])

        d0 = start_group(w, 0, 0)
        d1 = start_group(w, 1, 1)

        def pair_body(j, cc):
            d0.wait()
            reduce_group(rowb[0], 2 * j)
            start_group(w, 2 * j + 2, 0)
            d1.wait()
            reduce_group(rowb[1], 2 * j + 1)
            start_group(w, 2 * j + 3, 1)
            return cc

        lax.fori_loop(0, GW // 2 - 1, pair_body, 0)
        d0.wait()
        reduce_group(rowb[0], GW - 2)
        d1.wait()
        reduce_group(rowb[1], GW - 1)
        pltpu.sync_copy(acc_v, e_sum_o.at[pl.ds(base, ACC_N)])
        return carry

    lax.fori_loop(0, NWIN, win_body, 0)


def _sc_enc(ids_enc, e_emb, nei_pack):
    mesh = plsc.VectorSubcoreMesh(core_axis_name="c", subcore_axis_name="s")
    f = pl.kernel(
        _sc_enc_body,
        out_type=(
            jax.ShapeDtypeStruct((N_ENC, DIM), jnp.float32),
            jax.ShapeDtypeStruct((N_ENC, MAX_NEIGHBOR), jnp.int32),
        ),
        mesh=mesh,
        scratch_types=[
            pltpu.VMEM((GE * 16,), jnp.int32),
            pltpu.VMEM((GE * 16,), jnp.int32),
            pltpu.VMEM((GE * 16, DIM), jnp.float32),
            pltpu.VMEM((GE * 16, DIM), jnp.float32),
            pltpu.VMEM((ACC_N, DIM), jnp.float32),
            pltpu.VMEM((ACC_N, 128), jnp.int32),
            pltpu.VMEM((ACC_N, MAX_NEIGHBOR), jnp.int32),
            pltpu.VMEM((ACC_N,), jnp.int32),
            pltpu.SemaphoreType.DMA,
            pltpu.SemaphoreType.DMA,
        ],
    )
    return f(ids_enc, e_emb, nei_pack)


_ABLK = 512


def _enc_body(es_ref, rid_ref, er_ref, w_ref, b_ref, o_ref):
    rids = rid_ref[...]
    iot = lax.broadcasted_iota(jnp.int32, (_ABLK, CNT_R), 1)
    cnt = jnp.zeros((_ABLK, CNT_R), jnp.float32)
    for j in range(MAX_NEIGHBOR):
        cnt = cnt + (rids[:, j:j + 1] == iot).astype(jnp.float32)
    enc = jnp.dot(cnt, er_ref[...], preferred_element_type=jnp.float32)
    agg = (es_ref[...] + enc) * (1.0 / MAX_NEIGHBOR)
    z = jnp.tanh(jnp.dot(agg, w_ref[...], preferred_element_type=jnp.float32)
                 + b_ref[...])
    n = jnp.sqrt(jnp.sum(z * z, axis=1, keepdims=True))
    o_ref[...] = z / jnp.maximum(n, 1e-12)


def _enc_tc(e_sum, rids, enc_r_emb, w_enc, b_enc):
    grid = (N_ENC // _ABLK,)
    return pl.pallas_call(
        _enc_body,
        grid=grid,
        in_specs=[
            pl.BlockSpec((_ABLK, DIM), lambda i: (i, 0)),
            pl.BlockSpec((_ABLK, MAX_NEIGHBOR), lambda i: (i, 0)),
            pl.BlockSpec((CNT_R, DIM), lambda i: (0, 0)),
            pl.BlockSpec((DIM, DIM), lambda i: (0, 0)),
            pl.BlockSpec((1, DIM), lambda i: (0, 0)),
        ],
        out_specs=pl.BlockSpec((_ABLK, DIM), lambda i: (i, 0)),
        out_shape=jax.ShapeDtypeStruct((N_ENC, DIM), jnp.float32),
    )(e_sum, rids, enc_r_emb, w_enc, b_enc)


_BBLK = 256


def _nrm(x):
    n = jnp.sqrt(jnp.sum(x * x, axis=1, keepdims=True))
    return x / jnp.maximum(n, 1e-12)


def _score(h, r, t):
    return -jnp.sum(jnp.abs(h + r - t), axis=1)


def _loss_body(eh, et, ehn0, ehn1, etn0, etn1,
               rp, rn0, rn1,
               h2, r2, t2, hn20, hn21, rn20, rn21, tn20, tn21,
               o_ref):
    pos1 = _score(eh[...], _nrm(rp[...]), et[...])
    neg1 = (_score(ehn0[...], _nrm(rn0[...]), etn0[...])
            + _score(ehn1[...], _nrm(rn1[...]), etn1[...]))
    l1 = jnp.sum(jnp.maximum(neg1 - pos1 + MARGIN, 0.0))
    pos2 = _score(_nrm(h2[...]), _nrm(r2[...]), _nrm(t2[...]))
    neg2 = (_score(_nrm(hn20[...]), _nrm(rn20[...]), _nrm(tn20[...]))
            + _score(_nrm(hn21[...]), _nrm(rn21[...]), _nrm(tn21[...])))
    l2 = jnp.sum(jnp.maximum(neg2 - pos2 + MARGIN, 0.0))

    @pl.when(pl.program_id(0) == 0)
    def _():
        o_ref[...] = jnp.zeros_like(o_ref)

    o_ref[...] += (l1 + l2)[None, None]


def _loss_tc(enc_out, rows_r1, rows_t2):
    grid = (B // _BBLK,)
    nb = B // _BBLK

    def spec(off):
        return pl.BlockSpec((_BBLK, DIM), lambda i, off=off: (off * nb + i, 0))

    in_specs = ([spec(m) for m in range(6)]
                + [spec(m) for m in range(3)]
                + [spec(m) for m in range(9)])
    args = [enc_out] * 6 + [rows_r1] * 3 + [rows_t2] * 9
    return pl.pallas_call(
        _loss_body,
        grid=grid,
        in_specs=in_specs,
        out_specs=pl.BlockSpec((1, 1), lambda i: (0, 0)),
        out_shape=jax.ShapeDtypeStruct((1, 1), jnp.float32),
    )(*args)


def kernel(hp, rp, tp, hn, rn, tn, e_emb, r_emb, enc_r_emb, W_enc, b_enc,
           train_g, train_w, corr):
    del train_w, corr
    i32 = jnp.int32
    hp = hp.astype(i32); rp = rp.astype(i32); tp = tp.astype(i32)
    hn = hn.astype(i32); rn = rn.astype(i32); tn = tn.astype(i32)
    hn0, hn1 = hn[0::2], hn[1::2]
    rn0, rn1 = rn[0::2], rn[1::2]
    tn0, tn1 = tn[0::2], tn[1::2]

    ids_enc = jnp.concatenate([hp, tp, hn0, hn1, tn0, tn1])
    ids_t2 = jnp.concatenate([hp, rp, tp, hn0, hn1, rn0, rn1, tn0, tn1])
    ids_r1 = jnp.concatenate([rp, rn0, rn1])

    tg = train_g.astype(i32)
    nei_pack = jnp.pad(
        jnp.concatenate([tg[:, :, 1], tg[:, :, 0]], axis=1),
        ((0, 0), (0, 128 - 2 * MAX_NEIGHBOR)))
    e_sum, rids = _sc_enc(ids_enc, e_emb, nei_pack)
    rows_t2, rows_r1 = _sc_rows(ids_t2, ids_r1, e_emb, r_emb)

    enc_out = _enc_tc(e_sum, rids, enc_r_emb, W_enc,
                      b_enc.reshape(1, DIM))

    out = _loss_tc(enc_out, rows_r1, rows_t2)
    return out[0, 0]


# speedup vs baseline: 1.0021x; 1.0021x over previous
import functools

import jax
import jax.numpy as jnp
from jax import lax
from jax.experimental import pallas as pl
from jax.experimental.pallas import tpu as pltpu
from jax.experimental.pallas import tpu_sc as plsc

CNT_E = 50000
CNT_R = 256
DIM = 512
MAX_NEIGHBOR = 16
NUM_NEG = 2
B = 1024
MARGIN = 1.0

NW = 32
N_ENC = 3 * B * 2
N_T2 = 9 * B
N_R1 = 3 * B
ENC_PER = N_ENC // NW
T2_PER = N_T2 // NW
R1_PER = N_R1 // NW
CH_A = 48
GE = 4
ACC_N = 48
GW = ACC_N // GE
NWIN = ENC_PER // ACC_N


def _sc_rows_body(ids_t2, ids_r1, e_emb, r_emb, rows_t2_o, rows_r1_o,
                  idx0, idx1, row0, row1, sem0, sem1):
    wid = lax.axis_index("s") * 2 + lax.axis_index("c")
    idxb = (idx0, idx1)
    rowb = (row0, row1)
    semb = (sem0, sem1)
    t2_base = wid * T2_PER
    r1_base = wid * R1_PER
    chunks = []
    for c in range(T2_PER // CH_A):
        chunks.append((ids_t2, e_emb, rows_t2_o, t2_base + c * CH_A))
    for c in range(R1_PER // CH_A):
        chunks.append((ids_r1, r_emb, rows_r1_o, r1_base + c * CH_A))
    descs = []
    for c, (src_ids, table, dst, base) in enumerate(chunks):
        p = c & 1
        pltpu.sync_copy(src_ids.at[pl.ds(base, CH_A)], idxb[p])
        descs.append(pltpu.async_copy(table.at[idxb[p]], rowb[p], semb[p]))
        if c > 0:
            descs[c - 1].wait()
            _, _, pdst, pbase = chunks[c - 1]
            pltpu.sync_copy(rowb[1 - p], pdst.at[pl.ds(pbase, CH_A)])
    descs[-1].wait()
    _, _, pdst, pbase = chunks[-1]
    pltpu.sync_copy(rowb[(len(chunks) - 1) & 1], pdst.at[pl.ds(pbase, CH_A)])


def _sc_rows(ids_t2, ids_r1, e_emb, r_emb):
    mesh = plsc.VectorSubcoreMesh(core_axis_name="c", subcore_axis_name="s")
    f = pl.kernel(
        _sc_rows_body,
        out_type=(
            jax.ShapeDtypeStruct((N_T2, DIM), jnp.float32),
            jax.ShapeDtypeStruct((N_R1, DIM), jnp.float32),
        ),
        mesh=mesh,
        scratch_types=[
            pltpu.VMEM((CH_A,), jnp.int32),
            pltpu.VMEM((CH_A,), jnp.int32),
            pltpu.VMEM((CH_A, DIM), jnp.float32),
            pltpu.VMEM((CH_A, DIM), jnp.float32),
            pltpu.SemaphoreType.DMA,
            pltpu.SemaphoreType.DMA,
        ],
    )
    return f(ids_t2, ids_r1, e_emb, r_emb)


def _sc_enc_body(ids_enc, e_emb, nei_pack,
                 e_sum_o, rids_o,
                 idx0, idx1, row0, row1, acc_v, prow_v, rids_v, pidx_v,
                 sem0, sem1):
    wid = lax.axis_index("s") * 2 + lax.axis_index("c")
    idxb = (idx0, idx1)
    rowb = (row0, row1)
    semb = (sem0, sem1)
    enc_base = wid * ENC_PER


    def reduce_group(buf, g2):
        for k in range(GE):
            slot = g2 * GE + k

            def col_body(c, cc, k=k, slot=slot):
                vals = [buf[k * 16 + r, pl.ds(c * 16, 16)] for r in range(16)]
                while len(vals) > 1:
                    vals = [vals[i] + vals[i + 1]
                            for i in range(0, len(vals), 2)]
                acc_v[slot, pl.ds(c * 16, 16)] = vals[0]
                return cc

            lax.fori_loop(0, DIM // 16, col_body, 0)

    def start_group(w, g2, p):
        for k in range(GE):
            e = g2 * GE + k
            idxb[p][pl.ds(k * 16, 16)] = prow_v[e, pl.ds(0, 16)]
        return pltpu.async_copy(e_emb.at[idxb[p]], rowb[p], semb[p])

    def win_body(w, carry):
        base = enc_base + w * ACC_N
        pltpu.sync_copy(ids_enc.at[pl.ds(base, ACC_N)], pidx_v)
        pltpu.async_copy(nei_pack.at[pidx_v], prow_v, sem0).wait()

        def rid_body(e, c):
            rids_v[e] = prow_v[e, pl.ds(16, 16)]
            return c

        lax.fori_loop(0, ACC_N, rid_body, 0)
        pltpu.sync_copy(rids_v, rids_o.at[pl.ds(base, ACC_N)])

        d0 = start_group(w, 0, 0)
        d1 = start_group(w, 1, 1)

        def pair_body(j, cc):
            d0.wait()
            reduce_group(rowb[0], 2 * j)
            start_group(w, 2 * j + 2, 0)
            d1.wait()
            reduce_group(rowb[1], 2 * j + 1)
            start_group(w, 2 * j + 3, 1)
            return cc

        lax.fori_loop(0, GW // 2 - 1, pair_body, 0)
        d0.wait()
        reduce_group(rowb[0], GW - 2)
        d1.wait()
        reduce_group(rowb[1], GW - 1)
        pltpu.sync_copy(acc_v, e_sum_o.at[pl.ds(base, ACC_N)])
        return carry

    lax.fori_loop(0, NWIN, win_body, 0)


def _sc_enc(ids_enc, e_emb, nei_pack):
    mesh = plsc.VectorSubcoreMesh(core_axis_name="c", subcore_axis_name="s")
    f = pl.kernel(
        _sc_enc_body,
        out_type=(
            jax.ShapeDtypeStruct((N_ENC, DIM), jnp.float32),
            jax.ShapeDtypeStruct((N_ENC, MAX_NEIGHBOR), jnp.int32),
        ),
        mesh=mesh,
        scratch_types=[
            pltpu.VMEM((GE * 16,), jnp.int32),
            pltpu.VMEM((GE * 16,), jnp.int32),
            pltpu.VMEM((GE * 16, DIM), jnp.float32),
            pltpu.VMEM((GE * 16, DIM), jnp.float32),
            pltpu.VMEM((ACC_N, DIM), jnp.float32),
            pltpu.VMEM((ACC_N, 128), jnp.int32),
            pltpu.VMEM((ACC_N, MAX_NEIGHBOR), jnp.int32),
            pltpu.VMEM((ACC_N,), jnp.int32),
            pltpu.SemaphoreType.DMA,
            pltpu.SemaphoreType.DMA,
        ],
    )
    return f(ids_enc, e_emb, nei_pack)


_ABLK = 512


def _enc_body(es_ref, rid_ref, er_ref, w_ref, b_ref, o_ref):
    rids = rid_ref[...]
    iot = lax.broadcasted_iota(jnp.int32, (_ABLK, CNT_R), 1)
    cnt = jnp.zeros((_ABLK, CNT_R), jnp.float32)
    for j in range(MAX_NEIGHBOR):
        cnt = cnt + (rids[:, j:j + 1] == iot).astype(jnp.float32)
    enc = jnp.dot(cnt, er_ref[...], preferred_element_type=jnp.float32)
    agg = (es_ref[...] + enc) * (1.0 / MAX_NEIGHBOR)
    z = jnp.tanh(jnp.dot(agg, w_ref[...], preferred_element_type=jnp.float32)
                 + b_ref[...])
    n = jnp.sqrt(jnp.sum(z * z, axis=1, keepdims=True))
    o_ref[...] = z / jnp.maximum(n, 1e-12)


def _enc_tc(e_sum, rids, enc_r_emb, w_enc, b_enc):
    grid = (N_ENC // _ABLK,)
    return pl.pallas_call(
        _enc_body,
        grid=grid,
        in_specs=[
            pl.BlockSpec((_ABLK, DIM), lambda i: (i, 0)),
            pl.BlockSpec((_ABLK, MAX_NEIGHBOR), lambda i: (i, 0)),
            pl.BlockSpec((CNT_R, DIM), lambda i: (0, 0)),
            pl.BlockSpec((DIM, DIM), lambda i: (0, 0)),
            pl.BlockSpec((1, DIM), lambda i: (0, 0)),
        ],
        out_specs=pl.BlockSpec((_ABLK, DIM), lambda i: (i, 0)),
        out_shape=jax.ShapeDtypeStruct((N_ENC, DIM), jnp.float32),
    )(e_sum, rids, enc_r_emb, w_enc, b_enc)


_BBLK = 256


def _nrm(x):
    n = jnp.sqrt(jnp.sum(x * x, axis=1, keepdims=True))
    return x / jnp.maximum(n, 1e-12)


def _score(h, r, t):
    return -jnp.sum(jnp.abs(h + r - t), axis=1)


def _loss_body(eh, et, ehn0, ehn1, etn0, etn1,
               rp, rn0, rn1,
               h2, r2, t2, hn20, hn21, rn20, rn21, tn20, tn21,
               o_ref):
    pos1 = _score(eh[...], _nrm(rp[...]), et[...])
    neg1 = (_score(ehn0[...], _nrm(rn0[...]), etn0[...])
            + _score(ehn1[...], _nrm(rn1[...]), etn1[...]))
    l1 = jnp.sum(jnp.maximum(neg1 - pos1 + MARGIN, 0.0))
    pos2 = _score(_nrm(h2[...]), _nrm(r2[...]), _nrm(t2[...]))
    neg2 = (_score(_nrm(hn20[...]), _nrm(rn20[...]), _nrm(tn20[...]))
            + _score(_nrm(hn21[...]), _nrm(rn21[...]), _nrm(tn21[...])))
    l2 = jnp.sum(jnp.maximum(neg2 - pos2 + MARGIN, 0.0))

    @pl.when(pl.program_id(0) == 0)
    def _():
        o_ref[...] = jnp.zeros_like(o_ref)

    o_ref[...] += (l1 + l2)[None, None]


def _loss_tc(enc_out, rows_r1, rows_t2):
    grid = (B // _BBLK,)
    nb = B // _BBLK

    def spec(off):
        return pl.BlockSpec((_BBLK, DIM), lambda i, off=off: (off * nb + i, 0))

    in_specs = ([spec(m) for m in range(6)]
                + [spec(m) for m in range(3)]
                + [spec(m) for m in range(9)])
    args = [enc_out] * 6 + [rows_r1] * 3 + [rows_t2] * 9
    return pl.pallas_call(
        _loss_body,
        grid=grid,
        in_specs=in_specs,
        out_specs=pl.BlockSpec((1, 1), lambda i: (0, 0)),
        out_shape=jax.ShapeDtypeStruct((1, 1), jnp.float32),
    )(*args)


def kernel(hp, rp, tp, hn, rn, tn, e_emb, r_emb, enc_r_emb, W_enc, b_enc,
           train_g, train_w, corr):
    del train_w, corr
    i32 = jnp.int32
    hp = hp.astype(i32); rp = rp.astype(i32); tp = tp.astype(i32)
    hn = hn.astype(i32); rn = rn.astype(i32); tn = tn.astype(i32)
    hn0, hn1 = hn[0::2], hn[1::2]
    rn0, rn1 = rn[0::2], rn[1::2]
    tn0, tn1 = tn[0::2], tn[1::2]

    ids_enc = jnp.concatenate([hp, tp, hn0, hn1, tn0, tn1])
    ids_t2 = jnp.concatenate([hp, rp, tp, hn0, hn1, rn0, rn1, tn0, tn1])
    ids_r1 = jnp.concatenate([rp, rn0, rn1])

    tg = train_g.astype(i32)
    nei_pack = jnp.pad(
        jnp.concatenate([tg[:, :, 1], tg[:, :, 0]], axis=1),
        ((0, 0), (0, 128 - 2 * MAX_NEIGHBOR)))
    e_sum, rids = _sc_enc(ids_enc, e_emb, nei_pack)
    rows_t2, rows_r1 = _sc_rows(ids_t2, ids_r1, e_emb, r_emb)

    enc_out = _enc_tc(e_sum, rids, enc_r_emb, W_enc,
                      b_enc.reshape(1, DIM))

    out = _loss_tc(enc_out, rows_r1, rows_t2)
    return out[0, 0]
